# baseline (speedup 1.0000x reference)
import functools

import jax
import jax.numpy as jnp
from jax import lax
from jax.experimental import pallas as pl
from jax.experimental.pallas import tpu as pltpu
from jax.experimental.pallas import tpu_sc as plsc

D_M = 64
SCALE = 8.0
NC = 2
NS = 16
NW = NC * NS
G = 128
IB = 256


@functools.lru_cache(maxsize=None)
def _build_gather(S0, S1, V):
    IW = S0 // NW
    NQ = IW // G
    mesh = plsc.VectorSubcoreMesh(core_axis_name="c", subcore_axis_name="s")

    @functools.partial(
        pl.kernel,
        out_type=jax.ShapeDtypeStruct((S1, S0, D_M), jnp.float32),
        mesh=mesh,
        scratch_types=[
            pltpu.VMEM((NQ, G), jnp.int32),
            pltpu.VMEM((NQ, G), jnp.int32),
            pltpu.VMEM((NQ, G), jnp.int32),
            pltpu.VMEM((G, 2 * D_M), jnp.float32),
            pltpu.VMEM((G, D_M), jnp.float32),
            pltpu.SemaphoreType.DMA,
        ],
        compiler_params=pltpu.CompilerParams(use_tc_tiling_on_sc=True),
    )
    def k(xt_hbm, tbl_hbm, out_hbm, idx_v, pr_v, off_v, rows_v, sel_v, gsem):
        wid = lax.axis_index("s") * NC + lax.axis_index("c")
        i0 = wid * IW

        def j_body(j, carry):
            for q in range(NQ):
                pltpu.sync_copy(
                    xt_hbm.at[j, pl.ds(i0 + q * G, G)], idx_v.at[q]
                )

            def cvt_body(r, ccarry):
                for s in range(G // 16):
                    sl = (r, pl.ds(16 * s, 16))
                    v = idx_v[sl]
                    pr_v[sl] = jax.lax.shift_right_logical(v, 1)
                    off_v[sl] = (v & 1) * D_M
                return ccarry

            lax.fori_loop(0, NQ, cvt_body, 0)

            for q in range(NQ):
                pltpu.async_copy(
                    tbl_hbm.at[pr_v.at[q]], rows_v, gsem
                ).wait()

                def row_body(r, rcarry):
                    off = off_v[q, pl.ds(r, 1)][0]
                    for s in range(D_M // 16):
                        sel_v[r, pl.ds(16 * s, 16)] = rows_v[
                            r, pl.ds(off + 16 * s, 16)
                        ]
                    return rcarry

                lax.fori_loop(0, G, row_body, 0)
                pltpu.sync_copy(
                    sel_v, out_hbm.at[j, pl.ds(i0 + q * G, G)]
                )
            return carry

        lax.fori_loop(0, S1, j_body, 0)

    return k


def _tc_transpose_body(s_ref, o_ref):
    o_ref[...] = jnp.transpose(s_ref[...], (0, 2, 1)) + SCALE


@functools.lru_cache(maxsize=None)
def _build_transpose(S0, S1):
    return pl.pallas_call(
        _tc_transpose_body,
        out_shape=jax.ShapeDtypeStruct((S1, D_M, S0), jnp.float32),
        grid=(S1, S0 // IB),
        in_specs=[
            pl.BlockSpec((1, IB, D_M), lambda j, i: (j, i, 0)),
        ],
        out_specs=pl.BlockSpec((1, D_M, IB), lambda j, i: (j, 0, i)),
    )


def kernel(x, table):
    s0, s1 = x.shape
    V = table.shape[0]
    xt = x.T
    tbl2 = table.reshape(V // 2, 2 * D_M)
    scratch = _build_gather(s0, s1, V)(xt, tbl2)
    out = _build_transpose(s0, s1)(scratch)
    return out.transpose(2, 0, 1)


# speedup vs baseline: 3.4331x; 3.4331x over previous
import functools

import jax
import jax.numpy as jnp
from jax import lax
from jax.experimental import pallas as pl
from jax.experimental.pallas import tpu as pltpu
from jax.experimental.pallas import tpu_sc as plsc

D_M = 64
SCALE = 8.0
NC = 2
NS = 16
NW = NC * NS
G = 128
IB = 2048
RB = 4096


def _tc_pair_body(t_ref, o_ref):
    t = jnp.transpose(t_ref[...], (1, 0))
    o_ref[:, 0:D_M] = t
    o_ref[:, D_M:2 * D_M] = t


@functools.lru_cache(maxsize=None)
def _build_pair(V):
    return pl.pallas_call(
        _tc_pair_body,
        out_shape=jax.ShapeDtypeStruct((V, 2 * D_M), jnp.float32),
        grid=(pl.cdiv(V, RB),),
        in_specs=[pl.BlockSpec((D_M, RB), lambda i: (0, i))],
        out_specs=pl.BlockSpec((RB, 2 * D_M), lambda i: (i, 0)),
    )


@functools.lru_cache(maxsize=None)
def _build_gather(S0, S1, V):
    IW = S0 // NW
    NQ = IW // G
    mesh = plsc.VectorSubcoreMesh(core_axis_name="c", subcore_axis_name="s")

    @functools.partial(
        pl.kernel,
        out_type=jax.ShapeDtypeStruct((S1 // 2, S0, 2 * D_M), jnp.float32),
        mesh=mesh,
        scratch_types=[
            pltpu.VMEM((NQ, G), jnp.int32),
            pltpu.VMEM((NQ, G), jnp.int32),
            pltpu.VMEM((G, 2 * D_M), jnp.float32),
            pltpu.VMEM((G, 2 * D_M), jnp.float32),
            pltpu.VMEM((G, 2 * D_M), jnp.float32),
            pltpu.SemaphoreType.DMA,
        ],
    )
    def k(xt_hbm, tbl_hbm, out_hbm, ie_v, io_v, re_v, ro_v, big_v, gsem):
        wid = lax.axis_index("s") * NC + lax.axis_index("c")
        i0 = wid * IW

        def j_body(jp, carry):
            j = jp + jp
            for q in range(NQ):
                pltpu.sync_copy(
                    xt_hbm.at[j, pl.ds(i0 + q * G, G)], ie_v.at[q]
                )
                pltpu.sync_copy(
                    xt_hbm.at[j + 1, pl.ds(i0 + q * G, G)], io_v.at[q]
                )
            for q in range(NQ):
                pltpu.async_copy(tbl_hbm.at[ie_v.at[q]], re_v, gsem)
                cpo = pltpu.async_copy(tbl_hbm.at[io_v.at[q]], ro_v, gsem)
                cpo.wait()
                cpo.wait()

                def row_body(r, rcarry):
                    for s in range(D_M // 16):
                        sl = pl.ds(16 * s, 16)
                        big_v[r, pl.ds(16 * s, 16)] = re_v[r, sl]
                        big_v[r, pl.ds(D_M + 16 * s, 16)] = ro_v[r, sl]
                    return rcarry

                lax.fori_loop(0, G, row_body, 0)
                pltpu.sync_copy(
                    big_v, out_hbm.at[jp, pl.ds(i0 + q * G, G)]
                )
            return carry

        lax.fori_loop(0, S1 // 2, j_body, 0)

    return k


def _tc_transpose_body(s_ref, o_ref):
    t = jnp.transpose(s_ref[0], (1, 0))
    o_ref[...] = t.reshape(2, D_M, IB) + SCALE


@functools.lru_cache(maxsize=None)
def _build_transpose(S0, S1):
    return pl.pallas_call(
        _tc_transpose_body,
        out_shape=jax.ShapeDtypeStruct((S1, D_M, S0), jnp.float32),
        grid=(S1 // 2, S0 // IB),
        in_specs=[
            pl.BlockSpec((1, IB, 2 * D_M), lambda j, i: (j, i, 0)),
        ],
        out_specs=pl.BlockSpec((2, D_M, IB), lambda j, i: (j, 0, i)),
    )


def kernel(x, table):
    s0, s1 = x.shape
    V = table.shape[0]
    xt = x.T
    tblp = _build_pair(V)(table.T)
    scratch = _build_gather(s0, s1, V)(xt, tblp)
    out = _build_transpose(s0, s1)(scratch)
    return out.transpose(2, 0, 1)
